# baseline (speedup 1.0000x reference)
import functools

import jax
import jax.numpy as jnp
from jax import lax
from jax.experimental import pallas as pl
from jax.experimental.pallas import tpu as pltpu
from jax.experimental.pallas import tpu_sc as plsc

B = 16384
RD = 14
CD = 64
F = 4

TC_BLK = 1024
TC_GRID = B // TC_BLK


def _tc_dense(xt_ref, w_ref, b_ref, y_ref):
    y = lax.dot_general(xt_ref[...], w_ref[...],
                        dimension_numbers=(((0,), (0,)), ((), ())),
                        preferred_element_type=jnp.float32)
    bias = b_ref[...]
    y_ref[0] = y[:, :128] + bias[:, :128]
    y_ref[1] = y[:, 128:] + bias[:, 128:]


TP_BLK = 256
TP_GRID = (B // 2) // TP_BLK


def _tc_out(rs_ref, o_ref):
    x = rs_ref[...]
    xt = x.T
    ev = xt[:CD, :]
    od = xt[CD:, :]
    o_ref[...] = jnp.stack([ev, od], axis=-1).reshape(CD, 2 * TP_BLK)


NC = 2
NS = 16
L = 16
NW = NC * NS
CHUNK = B // NW
SEG = 128
NSEG = CHUNK // SEG


def _sc_route(yall_hbm, ids_hbm, out_hbm, ids_v, idx_v, rows_v, sem):
    wid = lax.axis_index("s") * NC + lax.axis_index("c")
    base = wid * CHUNK
    pltpu.sync_copy(ids_hbm.at[pl.ds(base, CHUNK)], ids_v)
    lane = lax.iota(jnp.int32, L)
    for s in range(NSEG):
        for j in range(SEG // L):
            off = s * SEG + j * L
            ids16 = ids_v[pl.ds(off, L)]
            idx_v[s, pl.ds(j * L, L)] = (
                (ids16 >> 1) * (2 * B) + 2 * (base + off) + 2 * lane
                + (ids16 & 1))
    copies = [
        pltpu.async_copy(yall_hbm.at[idx_v.at[s]],
                         rows_v.at[pl.ds(s * SEG, SEG)], sem)
        for s in range(NSEG)
    ]
    for c in copies:
        c.wait()
    pltpu.sync_copy(rows_v, out_hbm.at[pl.ds(base, CHUNK)])


def kernel(relation_logits, frame_type_ids, W, b):
    wall = W.transpose(0, 2, 1).transpose(1, 0, 2).reshape(RD, F * CD)
    ball = b.reshape(1, F * CD)
    xt = relation_logits.T

    yall = pl.pallas_call(
        _tc_dense,
        grid=(TC_GRID,),
        in_specs=[
            pl.BlockSpec((RD, TC_BLK), lambda i: (0, i)),
            pl.BlockSpec((RD, F * CD), lambda i: (0, 0)),
            pl.BlockSpec((1, F * CD), lambda i: (0, 0)),
        ],
        out_specs=pl.BlockSpec((2, TC_BLK, 2 * CD), lambda i: (0, i, 0)),
        out_shape=jax.ShapeDtypeStruct((2, B, 2 * CD), jnp.float32),
    )(xt, wall, ball)

    yall_rows = yall.reshape(F * B, CD)

    sc_call = functools.partial(
        pl.kernel,
        mesh=plsc.VectorSubcoreMesh(core_axis_name="c", subcore_axis_name="s"),
        compiler_params=pltpu.CompilerParams(use_tc_tiling_on_sc=False),
        out_type=jax.ShapeDtypeStruct((B, CD), jnp.float32),
        scratch_types=[
            pltpu.VMEM((CHUNK,), jnp.int32),
            pltpu.VMEM((NSEG, SEG), jnp.int32),
            pltpu.VMEM((CHUNK, CD), jnp.float32),
            pltpu.SemaphoreType.DMA,
        ],
    )(_sc_route)
    routed = sc_call(yall_rows, frame_type_ids)

    rs = routed.reshape(B // 2, 2 * CD)
    outT = pl.pallas_call(
        _tc_out,
        grid=(TP_GRID,),
        in_specs=[pl.BlockSpec((TP_BLK, 2 * CD), lambda i: (i, 0))],
        out_specs=pl.BlockSpec((CD, 2 * TP_BLK), lambda i: (0, i)),
        out_shape=jax.ShapeDtypeStruct((CD, B), jnp.float32),
    )(rs)
    return outT.T


# speedup vs baseline: 2.9243x; 2.9243x over previous
import functools

import jax
import jax.numpy as jnp
from jax import lax
from jax.experimental import pallas as pl
from jax.experimental.pallas import tpu as pltpu
from jax.experimental.pallas import tpu_sc as plsc

B = 16384
RD = 14
CD = 64
F = 4

TC_BLK = 1024
TC_GRID = B // TC_BLK


def _tc_dense(xt_ref, w_ref, b_ref, y_ref):
    y = lax.dot_general(xt_ref[...], w_ref[...],
                        dimension_numbers=(((0,), (0,)), ((), ())),
                        preferred_element_type=jnp.float32)
    bias = b_ref[...]
    y_ref[0] = y[:, :128] + bias[:, :128]
    y_ref[1] = y[:, 128:] + bias[:, 128:]


TP_BLK = 256
TP_GRID = (B // 2) // TP_BLK


def _tc_out(rs_ref, o_ref):
    x = rs_ref[...]
    xt = x.T
    o_ref[:, :TP_BLK] = xt[:CD, :]
    o_ref[:, TP_BLK:] = xt[CD:, :]


NC = 2
NS = 16
L = 16
NW = NC * NS
CHUNK = B // NW
SEG = 128
NSEG = CHUNK // SEG


def _sc_route(yall_hbm, ids_hbm, out_hbm, ids_v, idx_v, rows_v, sem):
    wid = lax.axis_index("s") * NC + lax.axis_index("c")
    base = wid * CHUNK
    pltpu.sync_copy(ids_hbm.at[pl.ds(base, CHUNK)], ids_v)
    lane = lax.iota(jnp.int32, L)
    for s in range(NSEG):
        for j in range(SEG // L):
            off = s * SEG + j * L
            ids16 = ids_v[pl.ds(off, L)]
            idx_v[s, pl.ds(j * L, L)] = (
                (ids16 >> 1) * (2 * B) + 2 * (base + off) + 2 * lane
                + (ids16 & 1))
    copies = [
        pltpu.async_copy(yall_hbm.at[idx_v.at[s]],
                         rows_v.at[pl.ds(s * SEG, SEG)], sem)
        for s in range(NSEG)
    ]
    for c in copies:
        c.wait()
    half = CHUNK // 2
    pltpu.sync_copy(rows_v.at[pl.ds(0, half)],
                    out_hbm.at[pl.ds(wid * half, half), 0])
    pltpu.sync_copy(rows_v.at[pl.ds(half, half)],
                    out_hbm.at[pl.ds(wid * half, half), 1])


def kernel(relation_logits, frame_type_ids, W, b):
    wall = W.transpose(0, 2, 1).transpose(1, 0, 2).reshape(RD, F * CD)
    ball = b.reshape(1, F * CD)
    xt = relation_logits.T

    yall = pl.pallas_call(
        _tc_dense,
        grid=(TC_GRID,),
        in_specs=[
            pl.BlockSpec((RD, TC_BLK), lambda i: (0, i)),
            pl.BlockSpec((RD, F * CD), lambda i: (0, 0)),
            pl.BlockSpec((1, F * CD), lambda i: (0, 0)),
        ],
        out_specs=pl.BlockSpec((2, TC_BLK, 2 * CD), lambda i: (0, i, 0)),
        out_shape=jax.ShapeDtypeStruct((2, B, 2 * CD), jnp.float32),
    )(xt, wall, ball)

    yall_rows = yall.reshape(F * B, CD)

    sc_call = functools.partial(
        pl.kernel,
        mesh=plsc.VectorSubcoreMesh(core_axis_name="c", subcore_axis_name="s"),
        compiler_params=pltpu.CompilerParams(use_tc_tiling_on_sc=False),
        out_type=jax.ShapeDtypeStruct((B // 2, 2, CD), jnp.float32),
        scratch_types=[
            pltpu.VMEM((CHUNK,), jnp.int32),
            pltpu.VMEM((NSEG, SEG), jnp.int32),
            pltpu.VMEM((CHUNK, CD), jnp.float32),
            pltpu.SemaphoreType.DMA,
        ],
    )(_sc_route)
    routed = sc_call(yall_rows, frame_type_ids)

    rs = routed.reshape(B // 2, 2 * CD)
    outT = pl.pallas_call(
        _tc_out,
        grid=(TP_GRID,),
        in_specs=[pl.BlockSpec((TP_BLK, 2 * CD), lambda i: (i, 0))],
        out_specs=pl.BlockSpec((CD, 2 * TP_BLK), lambda i: (0, i)),
        out_shape=jax.ShapeDtypeStruct((CD, B), jnp.float32),
    )(rs)
    return outT.T


# speedup vs baseline: 4.5724x; 1.5636x over previous
import functools

import jax
import jax.numpy as jnp
from jax import lax
from jax.experimental import pallas as pl
from jax.experimental.pallas import tpu as pltpu
from jax.experimental.pallas import tpu_sc as plsc

B = 16384
RD = 14
CD = 64
F = 4

TC_BLK = 1024
TC_GRID = B // TC_BLK


def _tc_dense(xt_ref, w_ref, b_ref, y_ref):
    y = lax.dot_general(xt_ref[...], w_ref[...],
                        dimension_numbers=(((0,), (0,)), ((), ())),
                        preferred_element_type=jnp.float32)
    bias = b_ref[...]
    y_ref[0] = y[:, :128] + bias[:, :128]
    y_ref[1] = y[:, 128:] + bias[:, 128:]


TP_BLK = 256
TP_GRID = (B // 2) // TP_BLK


def _tc_out(rs_ref, o_ref):
    x = rs_ref[...]
    xt = x.T
    o_ref[:, :TP_BLK] = xt[:CD, :]
    o_ref[:, TP_BLK:] = xt[CD:, :]


NC = 2
NS = 16
L = 16
NW = NC * NS
CHUNK = B // NW
SEG = 128
NSEG = CHUNK // SEG


def _sc_route(yall_hbm, ids_hbm, out_hbm, ids_v, idx_v, rows_v, sem):
    wid = lax.axis_index("s") * NC + lax.axis_index("c")
    base = wid * CHUNK
    pltpu.sync_copy(ids_hbm.at[pl.ds(base, CHUNK)], ids_v)
    lane = lax.iota(jnp.int32, L)
    for s in range(NSEG):
        for j in range(SEG // L):
            off = s * SEG + j * L
            ids16 = ids_v[pl.ds(off, L)]
            idx_v[s, pl.ds(j * L, L)] = (
                (ids16 >> 1) * (2 * B) + 2 * (base + off) + 2 * lane
                + (ids16 & 1))
    copies = [
        pltpu.async_copy(yall_hbm.at[idx_v.at[s]],
                         rows_v.at[pl.ds(s * SEG, SEG)], sem)
        for s in range(NSEG)
    ]
    for c in copies:
        c.wait()
    half = CHUNK // 2
    pltpu.sync_copy(rows_v.at[pl.ds(0, half)],
                    out_hbm.at[pl.ds(wid * half, half), pl.ds(0, CD)])
    pltpu.sync_copy(rows_v.at[pl.ds(half, half)],
                    out_hbm.at[pl.ds(wid * half, half), pl.ds(CD, CD)])


def kernel(relation_logits, frame_type_ids, W, b):
    wall = W.transpose(0, 2, 1).transpose(1, 0, 2).reshape(RD, F * CD)
    ball = b.reshape(1, F * CD)
    xt = relation_logits.T

    yall = pl.pallas_call(
        _tc_dense,
        grid=(TC_GRID,),
        in_specs=[
            pl.BlockSpec((RD, TC_BLK), lambda i: (0, i)),
            pl.BlockSpec((RD, F * CD), lambda i: (0, 0)),
            pl.BlockSpec((1, F * CD), lambda i: (0, 0)),
        ],
        out_specs=pl.BlockSpec((2, TC_BLK, 2 * CD), lambda i: (0, i, 0)),
        out_shape=jax.ShapeDtypeStruct((2, B, 2 * CD), jnp.float32),
    )(xt, wall, ball)

    yall_rows = yall.reshape(F * B, CD)

    sc_call = functools.partial(
        pl.kernel,
        mesh=plsc.VectorSubcoreMesh(core_axis_name="c", subcore_axis_name="s"),
        compiler_params=pltpu.CompilerParams(use_tc_tiling_on_sc=False),
        out_type=jax.ShapeDtypeStruct((B // 2, 2 * CD), jnp.float32),
        scratch_types=[
            pltpu.VMEM((CHUNK,), jnp.int32),
            pltpu.VMEM((NSEG, SEG), jnp.int32),
            pltpu.VMEM((CHUNK, CD), jnp.float32),
            pltpu.SemaphoreType.DMA,
        ],
    )(_sc_route)
    rs = sc_call(yall_rows, frame_type_ids)
    outT = pl.pallas_call(
        _tc_out,
        grid=(TP_GRID,),
        in_specs=[pl.BlockSpec((TP_BLK, 2 * CD), lambda i: (i, 0))],
        out_specs=pl.BlockSpec((CD, 2 * TP_BLK), lambda i: (0, i)),
        out_shape=jax.ShapeDtypeStruct((CD, B), jnp.float32),
    )(rs)
    return outT.T


# speedup vs baseline: 5.7798x; 1.2640x over previous
import functools

import jax
import jax.numpy as jnp
from jax import lax
from jax.experimental import pallas as pl
from jax.experimental.pallas import tpu as pltpu
from jax.experimental.pallas import tpu_sc as plsc

B = 16384
RD = 14
CD = 64
F = 4

TC_BLK = 1024
TC_GRID = B // TC_BLK


def _tc_dense(xt_ref, w_ref, b_ref, y_ref):
    y = lax.dot_general(xt_ref[...], w_ref[...],
                        dimension_numbers=(((0,), (0,)), ((), ())),
                        preferred_element_type=jnp.float32)
    bias = b_ref[...]
    y_ref[0] = y[:, :128] + bias[:, :128]
    y_ref[1] = y[:, 128:] + bias[:, 128:]


TP_BLK = 1024
TP_GRID = (B // 2) // TP_BLK


def _tc_out(rs_ref, o_ref):
    x = rs_ref[...]
    xt = x.T
    for k in range(TP_BLK // 256):
        o_ref[:, 512 * k:512 * k + 256] = xt[:CD, 256 * k:256 * k + 256]
        o_ref[:, 512 * k + 256:512 * k + 512] = xt[CD:, 256 * k:256 * k + 256]


NC = 2
NS = 16
L = 16
NW = NC * NS
CHUNK = B // NW
SEG = 128
NSEG = CHUNK // SEG


def _sc_route(yall_hbm, ids_hbm, out_hbm, ids_v, idx_v, rows_v, sem):
    wid = lax.axis_index("s") * NC + lax.axis_index("c")
    base = wid * CHUNK
    pltpu.sync_copy(ids_hbm.at[pl.ds(base, CHUNK)], ids_v)
    lane = lax.iota(jnp.int32, L)
    for s in range(NSEG):
        for j in range(SEG // L):
            off = s * SEG + j * L
            ids16 = ids_v[pl.ds(off, L)]
            idx_v[s, pl.ds(j * L, L)] = (
                (ids16 >> 1) * (2 * B) + 2 * (base + off) + 2 * lane
                + (ids16 & 1))
    copies = [
        pltpu.async_copy(yall_hbm.at[idx_v.at[s]],
                         rows_v.at[pl.ds(s * SEG, SEG)], sem)
        for s in range(NSEG)
    ]
    for c in copies:
        c.wait()
    half = CHUNK // 2
    pltpu.sync_copy(rows_v.at[pl.ds(0, half)],
                    out_hbm.at[pl.ds(wid * half, half), pl.ds(0, CD)])
    pltpu.sync_copy(rows_v.at[pl.ds(half, half)],
                    out_hbm.at[pl.ds(wid * half, half), pl.ds(CD, CD)])


def kernel(relation_logits, frame_type_ids, W, b):
    wall = W.transpose(0, 2, 1).transpose(1, 0, 2).reshape(RD, F * CD)
    ball = b.reshape(1, F * CD)
    xt = relation_logits.T

    yall = pl.pallas_call(
        _tc_dense,
        grid=(TC_GRID,),
        in_specs=[
            pl.BlockSpec((RD, TC_BLK), lambda i: (0, i)),
            pl.BlockSpec((RD, F * CD), lambda i: (0, 0)),
            pl.BlockSpec((1, F * CD), lambda i: (0, 0)),
        ],
        out_specs=pl.BlockSpec((2, TC_BLK, 2 * CD), lambda i: (0, i, 0)),
        out_shape=jax.ShapeDtypeStruct((2, B, 2 * CD), jnp.float32),
    )(xt, wall, ball)

    yall_rows = yall.reshape(F * B, CD)

    sc_call = functools.partial(
        pl.kernel,
        mesh=plsc.VectorSubcoreMesh(core_axis_name="c", subcore_axis_name="s"),
        compiler_params=pltpu.CompilerParams(use_tc_tiling_on_sc=False),
        out_type=jax.ShapeDtypeStruct((B // 2, 2 * CD), jnp.float32),
        scratch_types=[
            pltpu.VMEM((CHUNK,), jnp.int32),
            pltpu.VMEM((NSEG, SEG), jnp.int32),
            pltpu.VMEM((CHUNK, CD), jnp.float32),
            pltpu.SemaphoreType.DMA,
        ],
    )(_sc_route)
    rs = sc_call(yall_rows, frame_type_ids)
    outT = pl.pallas_call(
        _tc_out,
        grid=(TP_GRID,),
        in_specs=[pl.BlockSpec((TP_BLK, 2 * CD), lambda i: (i, 0))],
        out_specs=pl.BlockSpec((CD, 2 * TP_BLK), lambda i: (0, i)),
        out_shape=jax.ShapeDtypeStruct((CD, B), jnp.float32),
    )(rs)
    return outT.T


# speedup vs baseline: 6.4002x; 1.1073x over previous
import functools

import jax
import jax.numpy as jnp
from jax import lax
from jax.experimental import pallas as pl
from jax.experimental.pallas import tpu as pltpu
from jax.experimental.pallas import tpu_sc as plsc

B = 16384
RD = 14
CD = 64
F = 4

TC_BLK = 2048
TC_GRID = B // TC_BLK


def _tc_dense(xt_ref, w_ref, b_ref, y_ref):
    y = lax.dot_general(xt_ref[...], w_ref[...],
                        dimension_numbers=(((0,), (0,)), ((), ())),
                        preferred_element_type=jnp.float32)
    bias = b_ref[...]
    y_ref[0] = y[:, :128] + bias[:, :128]
    y_ref[1] = y[:, 128:] + bias[:, 128:]


TP_BLK = 1024
TP_GRID = (B // 2) // TP_BLK


def _tc_out(rs_ref, o_ref):
    x = rs_ref[...]
    xt = x.T
    for k in range(TP_BLK // 256):
        o_ref[:, 512 * k:512 * k + 256] = xt[:CD, 256 * k:256 * k + 256]
        o_ref[:, 512 * k + 256:512 * k + 512] = xt[CD:, 256 * k:256 * k + 256]


NC = 2
NS = 16
L = 16
NW = NC * NS
CHUNK = B // NW
SEG = 128
NSEG = CHUNK // SEG


def _sc_route(yall_hbm, ids_hbm, out_hbm, ids_v, idx_v, rows_v, sem):
    wid = lax.axis_index("s") * NC + lax.axis_index("c")
    base = wid * CHUNK
    pltpu.sync_copy(ids_hbm.at[pl.ds(base, CHUNK)], ids_v)
    lane = lax.iota(jnp.int32, L)
    for s in range(NSEG):
        for j in range(SEG // L):
            off = s * SEG + j * L
            ids16 = ids_v[pl.ds(off, L)]
            idx_v[s, pl.ds(j * L, L)] = (
                (ids16 >> 1) * (2 * B) + 2 * (base + off) + 2 * lane
                + (ids16 & 1))
    copies = [
        pltpu.async_copy(yall_hbm.at[idx_v.at[s]],
                         rows_v.at[pl.ds(s * SEG, SEG)], sem)
        for s in range(NSEG)
    ]
    for c in copies:
        c.wait()
    half = CHUNK // 2
    pltpu.sync_copy(rows_v.at[pl.ds(0, half)],
                    out_hbm.at[pl.ds(wid * half, half), pl.ds(0, CD)])
    pltpu.sync_copy(rows_v.at[pl.ds(half, half)],
                    out_hbm.at[pl.ds(wid * half, half), pl.ds(CD, CD)])


def kernel(relation_logits, frame_type_ids, W, b):
    wall = W.transpose(0, 2, 1).transpose(1, 0, 2).reshape(RD, F * CD)
    ball = b.reshape(1, F * CD)
    xt = relation_logits.T

    yall = pl.pallas_call(
        _tc_dense,
        grid=(TC_GRID,),
        in_specs=[
            pl.BlockSpec((RD, TC_BLK), lambda i: (0, i)),
            pl.BlockSpec((RD, F * CD), lambda i: (0, 0)),
            pl.BlockSpec((1, F * CD), lambda i: (0, 0)),
        ],
        out_specs=pl.BlockSpec((2, TC_BLK, 2 * CD), lambda i: (0, i, 0)),
        out_shape=jax.ShapeDtypeStruct((2, B, 2 * CD), jnp.float32),
    )(xt, wall, ball)

    yall_rows = yall.reshape(F * B, CD)

    sc_call = functools.partial(
        pl.kernel,
        mesh=plsc.VectorSubcoreMesh(core_axis_name="c", subcore_axis_name="s"),
        compiler_params=pltpu.CompilerParams(use_tc_tiling_on_sc=False),
        out_type=jax.ShapeDtypeStruct((B // 2, 2 * CD), jnp.float32),
        scratch_types=[
            pltpu.VMEM((CHUNK,), jnp.int32),
            pltpu.VMEM((NSEG, SEG), jnp.int32),
            pltpu.VMEM((CHUNK, CD), jnp.float32),
            pltpu.SemaphoreType.DMA,
        ],
    )(_sc_route)
    rs = sc_call(yall_rows, frame_type_ids)
    outT = pl.pallas_call(
        _tc_out,
        grid=(TP_GRID,),
        in_specs=[pl.BlockSpec((TP_BLK, 2 * CD), lambda i: (i, 0))],
        out_specs=pl.BlockSpec((CD, 2 * TP_BLK), lambda i: (0, i)),
        out_shape=jax.ShapeDtypeStruct((CD, B), jnp.float32),
    )(rs)
    return outT.T


# speedup vs baseline: 6.9875x; 1.0918x over previous
import functools

import jax
import jax.numpy as jnp
from jax import lax
from jax.experimental import pallas as pl
from jax.experimental.pallas import tpu as pltpu
from jax.experimental.pallas import tpu_sc as plsc

B = 16384
RD = 14
CD = 64
F = 4

TC_BLK = 4096
TC_GRID = B // TC_BLK


def _tc_dense(xt_ref, w_ref, b_ref, y_ref):
    y = lax.dot_general(xt_ref[...], w_ref[...],
                        dimension_numbers=(((0,), (0,)), ((), ())),
                        preferred_element_type=jnp.float32)
    bias = b_ref[...]
    y_ref[0] = y[:, :128] + bias[:, :128]
    y_ref[1] = y[:, 128:] + bias[:, 128:]


TP_BLK = 2048
TP_GRID = (B // 2) // TP_BLK


def _tc_out(rs_ref, o_ref):
    x = rs_ref[...]
    xt = x.T
    for k in range(TP_BLK // 256):
        o_ref[:, 512 * k:512 * k + 256] = xt[:CD, 256 * k:256 * k + 256]
        o_ref[:, 512 * k + 256:512 * k + 512] = xt[CD:, 256 * k:256 * k + 256]


NC = 2
NS = 16
L = 16
NW = NC * NS
CHUNK = B // NW
SEG = 128
NSEG = CHUNK // SEG


def _sc_route(yall_hbm, ids_hbm, out_hbm, ids_v, idx_v, rows_v, sem):
    wid = lax.axis_index("s") * NC + lax.axis_index("c")
    base = wid * CHUNK
    pltpu.sync_copy(ids_hbm.at[pl.ds(base, CHUNK)], ids_v)
    lane = lax.iota(jnp.int32, L)
    for s in range(NSEG):
        for j in range(SEG // L):
            off = s * SEG + j * L
            ids16 = ids_v[pl.ds(off, L)]
            idx_v[s, pl.ds(j * L, L)] = (
                (ids16 >> 1) * (2 * B) + 2 * (base + off) + 2 * lane
                + (ids16 & 1))
    copies = [
        pltpu.async_copy(yall_hbm.at[idx_v.at[s]],
                         rows_v.at[pl.ds(s * SEG, SEG)], sem)
        for s in range(NSEG)
    ]
    for c in copies:
        c.wait()
    half = CHUNK // 2
    pltpu.sync_copy(rows_v.at[pl.ds(0, half)],
                    out_hbm.at[pl.ds(wid * half, half), pl.ds(0, CD)])
    pltpu.sync_copy(rows_v.at[pl.ds(half, half)],
                    out_hbm.at[pl.ds(wid * half, half), pl.ds(CD, CD)])


def kernel(relation_logits, frame_type_ids, W, b):
    wall = W.transpose(0, 2, 1).transpose(1, 0, 2).reshape(RD, F * CD)
    ball = b.reshape(1, F * CD)
    xt = relation_logits.T

    yall = pl.pallas_call(
        _tc_dense,
        grid=(TC_GRID,),
        in_specs=[
            pl.BlockSpec((RD, TC_BLK), lambda i: (0, i)),
            pl.BlockSpec((RD, F * CD), lambda i: (0, 0)),
            pl.BlockSpec((1, F * CD), lambda i: (0, 0)),
        ],
        out_specs=pl.BlockSpec((2, TC_BLK, 2 * CD), lambda i: (0, i, 0)),
        out_shape=jax.ShapeDtypeStruct((2, B, 2 * CD), jnp.float32),
    )(xt, wall, ball)

    yall_rows = yall.reshape(F * B, CD)

    sc_call = functools.partial(
        pl.kernel,
        mesh=plsc.VectorSubcoreMesh(core_axis_name="c", subcore_axis_name="s"),
        compiler_params=pltpu.CompilerParams(use_tc_tiling_on_sc=False),
        out_type=jax.ShapeDtypeStruct((B // 2, 2 * CD), jnp.float32),
        scratch_types=[
            pltpu.VMEM((CHUNK,), jnp.int32),
            pltpu.VMEM((NSEG, SEG), jnp.int32),
            pltpu.VMEM((CHUNK, CD), jnp.float32),
            pltpu.SemaphoreType.DMA,
        ],
    )(_sc_route)
    rs = sc_call(yall_rows, frame_type_ids)
    outT = pl.pallas_call(
        _tc_out,
        grid=(TP_GRID,),
        in_specs=[pl.BlockSpec((TP_BLK, 2 * CD), lambda i: (i, 0))],
        out_specs=pl.BlockSpec((CD, 2 * TP_BLK), lambda i: (0, i)),
        out_shape=jax.ShapeDtypeStruct((CD, B), jnp.float32),
    )(rs)
    return outT.T


# speedup vs baseline: 7.2153x; 1.0326x over previous
import functools

import jax
import jax.numpy as jnp
from jax import lax
from jax.experimental import pallas as pl
from jax.experimental.pallas import tpu as pltpu
from jax.experimental.pallas import tpu_sc as plsc

B = 16384
RD = 14
CD = 64
F = 4

TC_BLK = 8192
TC_GRID = B // TC_BLK


def _tc_dense(xt_ref, w_ref, b_ref, y_ref):
    y = lax.dot_general(xt_ref[...], w_ref[...],
                        dimension_numbers=(((0,), (0,)), ((), ())),
                        preferred_element_type=jnp.float32)
    bias = b_ref[...]
    y_ref[0] = y[:, :128] + bias[:, :128]
    y_ref[1] = y[:, 128:] + bias[:, 128:]


TP_BLK = 4096
TP_GRID = (B // 2) // TP_BLK


def _tc_out(rs_ref, o_ref):
    x = rs_ref[...]
    xt = x.T
    for k in range(TP_BLK // 256):
        o_ref[:, 512 * k:512 * k + 256] = xt[:CD, 256 * k:256 * k + 256]
        o_ref[:, 512 * k + 256:512 * k + 512] = xt[CD:, 256 * k:256 * k + 256]


NC = 2
NS = 16
L = 16
NW = NC * NS
CHUNK = B // NW
SEG = 128
NSEG = CHUNK // SEG


def _sc_route(yall_hbm, ids_hbm, out_hbm, ids_v, idx_v, rows_v, sem):
    wid = lax.axis_index("s") * NC + lax.axis_index("c")
    base = wid * CHUNK
    pltpu.sync_copy(ids_hbm.at[pl.ds(base, CHUNK)], ids_v)
    lane = lax.iota(jnp.int32, L)
    for s in range(NSEG):
        for j in range(SEG // L):
            off = s * SEG + j * L
            ids16 = ids_v[pl.ds(off, L)]
            idx_v[s, pl.ds(j * L, L)] = (
                (ids16 >> 1) * (2 * B) + 2 * (base + off) + 2 * lane
                + (ids16 & 1))
    copies = [
        pltpu.async_copy(yall_hbm.at[idx_v.at[s]],
                         rows_v.at[pl.ds(s * SEG, SEG)], sem)
        for s in range(NSEG)
    ]
    for c in copies:
        c.wait()
    half = CHUNK // 2
    pltpu.sync_copy(rows_v.at[pl.ds(0, half)],
                    out_hbm.at[pl.ds(wid * half, half), pl.ds(0, CD)])
    pltpu.sync_copy(rows_v.at[pl.ds(half, half)],
                    out_hbm.at[pl.ds(wid * half, half), pl.ds(CD, CD)])


def kernel(relation_logits, frame_type_ids, W, b):
    wall = W.transpose(0, 2, 1).transpose(1, 0, 2).reshape(RD, F * CD)
    ball = b.reshape(1, F * CD)
    xt = relation_logits.T

    yall = pl.pallas_call(
        _tc_dense,
        grid=(TC_GRID,),
        in_specs=[
            pl.BlockSpec((RD, TC_BLK), lambda i: (0, i)),
            pl.BlockSpec((RD, F * CD), lambda i: (0, 0)),
            pl.BlockSpec((1, F * CD), lambda i: (0, 0)),
        ],
        out_specs=pl.BlockSpec((2, TC_BLK, 2 * CD), lambda i: (0, i, 0)),
        out_shape=jax.ShapeDtypeStruct((2, B, 2 * CD), jnp.float32),
    )(xt, wall, ball)

    yall_rows = yall.reshape(F * B, CD)

    sc_call = functools.partial(
        pl.kernel,
        mesh=plsc.VectorSubcoreMesh(core_axis_name="c", subcore_axis_name="s"),
        compiler_params=pltpu.CompilerParams(use_tc_tiling_on_sc=False),
        out_type=jax.ShapeDtypeStruct((B // 2, 2 * CD), jnp.float32),
        scratch_types=[
            pltpu.VMEM((CHUNK,), jnp.int32),
            pltpu.VMEM((NSEG, SEG), jnp.int32),
            pltpu.VMEM((CHUNK, CD), jnp.float32),
            pltpu.SemaphoreType.DMA,
        ],
    )(_sc_route)
    rs = sc_call(yall_rows, frame_type_ids)
    outT = pl.pallas_call(
        _tc_out,
        grid=(TP_GRID,),
        in_specs=[pl.BlockSpec((TP_BLK, 2 * CD), lambda i: (i, 0))],
        out_specs=pl.BlockSpec((CD, 2 * TP_BLK), lambda i: (0, i)),
        out_shape=jax.ShapeDtypeStruct((CD, B), jnp.float32),
    )(rs)
    return outT.T


# speedup vs baseline: 7.2391x; 1.0033x over previous
import functools

import jax
import jax.numpy as jnp
from jax import lax
from jax.experimental import pallas as pl
from jax.experimental.pallas import tpu as pltpu
from jax.experimental.pallas import tpu_sc as plsc

B = 16384
RD = 14
CD = 64
F = 4

TC_BLK = 8192
TC_GRID = B // TC_BLK


def _tc_dense(xt_ref, w_ref, b_ref, ids_ref, y_ref, idx_ref):
    i = pl.program_id(0)
    y = lax.dot_general(xt_ref[...], w_ref[...],
                        dimension_numbers=(((0,), (0,)), ((), ())),
                        preferred_element_type=jnp.float32)
    bias = b_ref[...]
    y_ref[0] = y[:, :128] + bias[:, :128]
    y_ref[1] = y[:, 128:] + bias[:, 128:]
    ids = ids_ref[...]
    rows = TC_BLK // 128
    t = (i * TC_BLK
         + lax.broadcasted_iota(jnp.int32, (rows, 128), 0) * 128
         + lax.broadcasted_iota(jnp.int32, (rows, 128), 1))
    idx_ref[...] = (ids >> 1) * (2 * B) + 2 * t + (ids & 1)


TP_BLK = 4096
TP_GRID = (B // 2) // TP_BLK


def _tc_out(rs_ref, o_ref):
    x = rs_ref[...]
    xt = x.T
    for k in range(TP_BLK // 256):
        o_ref[:, 512 * k:512 * k + 256] = xt[:CD, 256 * k:256 * k + 256]
        o_ref[:, 512 * k + 256:512 * k + 512] = xt[CD:, 256 * k:256 * k + 256]


NC = 2
NS = 16
L = 16
NW = NC * NS
CHUNK = B // NW
SEG = 128
NSEG = CHUNK // SEG


def _sc_route(yall_hbm, idx_hbm, out_hbm, idx_v, rows_v, sem):
    wid = lax.axis_index("s") * NC + lax.axis_index("c")
    base = wid * CHUNK
    pltpu.sync_copy(idx_hbm.at[pl.ds(base, CHUNK)], idx_v)
    copies = [
        pltpu.async_copy(yall_hbm.at[idx_v.at[pl.ds(s * SEG, SEG)]],
                         rows_v.at[pl.ds(s * SEG, SEG)], sem)
        for s in range(NSEG)
    ]
    for c in copies:
        c.wait()
    half = CHUNK // 2
    pltpu.sync_copy(rows_v.at[pl.ds(0, half)],
                    out_hbm.at[pl.ds(wid * half, half), pl.ds(0, CD)])
    pltpu.sync_copy(rows_v.at[pl.ds(half, half)],
                    out_hbm.at[pl.ds(wid * half, half), pl.ds(CD, CD)])


def kernel(relation_logits, frame_type_ids, W, b):
    wall = W.transpose(0, 2, 1).transpose(1, 0, 2).reshape(RD, F * CD)
    ball = b.reshape(1, F * CD)
    xt = relation_logits.T

    ids2d = frame_type_ids.reshape(B // 128, 128)

    yall, idx2d = pl.pallas_call(
        _tc_dense,
        grid=(TC_GRID,),
        in_specs=[
            pl.BlockSpec((RD, TC_BLK), lambda i: (0, i)),
            pl.BlockSpec((RD, F * CD), lambda i: (0, 0)),
            pl.BlockSpec((1, F * CD), lambda i: (0, 0)),
            pl.BlockSpec((TC_BLK // 128, 128), lambda i: (i, 0)),
        ],
        out_specs=[
            pl.BlockSpec((2, TC_BLK, 2 * CD), lambda i: (0, i, 0)),
            pl.BlockSpec((TC_BLK // 128, 128), lambda i: (i, 0)),
        ],
        out_shape=[
            jax.ShapeDtypeStruct((2, B, 2 * CD), jnp.float32),
            jax.ShapeDtypeStruct((B // 128, 128), jnp.int32),
        ],
    )(xt, wall, ball, ids2d)

    yall_rows = yall.reshape(F * B, CD)
    idx_flat = idx2d.reshape(B)

    sc_call = functools.partial(
        pl.kernel,
        mesh=plsc.VectorSubcoreMesh(core_axis_name="c", subcore_axis_name="s"),
        compiler_params=pltpu.CompilerParams(use_tc_tiling_on_sc=False),
        out_type=jax.ShapeDtypeStruct((B // 2, 2 * CD), jnp.float32),
        scratch_types=[
            pltpu.VMEM((CHUNK,), jnp.int32),
            pltpu.VMEM((CHUNK, CD), jnp.float32),
            pltpu.SemaphoreType.DMA,
        ],
    )(_sc_route)
    rs = sc_call(yall_rows, idx_flat)
    outT = pl.pallas_call(
        _tc_out,
        grid=(TP_GRID,),
        in_specs=[pl.BlockSpec((TP_BLK, 2 * CD), lambda i: (i, 0))],
        out_specs=pl.BlockSpec((CD, 2 * TP_BLK), lambda i: (0, i)),
        out_shape=jax.ShapeDtypeStruct((CD, B), jnp.float32),
    )(rs)
    return outT.T
